# baseline (speedup 1.0000x reference)
import functools

import jax
import jax.numpy as jnp
from jax import lax
from jax.experimental import pallas as pl
from jax.experimental.pallas import tpu as pltpu
from jax.experimental.pallas import tpu_sc as plsc

_G = 128
_K = 20


@functools.lru_cache(maxsize=None)
def _make_gather(num_groups, vocab, dim):
  info = plsc.get_sparse_core_info()
  nc, ns = info.num_cores, info.num_subcores
  nw = nc * ns
  groups_per_w = num_groups // nw
  outer = groups_per_w // _K
  assert groups_per_w % _K == 0

  mesh = plsc.VectorSubcoreMesh(core_axis_name="c", subcore_axis_name="s")

  @functools.partial(
      pl.kernel,
      mesh=mesh,
      out_type=jax.ShapeDtypeStruct((num_groups, _G, dim), jnp.float32),
      scratch_types=[
          pltpu.VMEM((_K, _G), jnp.int32),
          pltpu.VMEM((_K, _G, dim), jnp.float32),
          pltpu.SemaphoreType.DMA,
      ],
      compiler_params=pltpu.CompilerParams(use_tc_tiling_on_sc=False),
  )
  def gather_kernel(idx_hbm, table_hbm, out_hbm, idx_v, rows_v, sem):
    wid = lax.axis_index("s") * nc + lax.axis_index("c")
    tbase = wid * outer

    def step(o, carry):
      t = tbase + o
      pltpu.sync_copy(idx_hbm.at[t], idx_v)
      descs = []
      for j in range(_K):
        descs.append(
            pltpu.async_copy(table_hbm.at[idx_v.at[j]], rows_v.at[j], sem))
      for d in descs:
        d.wait()
      pltpu.sync_copy(rows_v, out_hbm.at[pl.ds(t * _K, _K)])
      return carry

    lax.fori_loop(0, outer, step, 0)

  return gather_kernel


def _mm_body(nl, x_ref, w_ref, b_ref, o_ref):
  bblk = x_ref.shape[0]
  dout = w_ref.shape[1]
  for t in range(0, nl, 8):
    r = min(8, nl - t)
    xt = x_ref[:, t:t + r, :].reshape(bblk * r, x_ref.shape[2])
    z = lax.dot_general(
        xt, w_ref[...],
        dimension_numbers=(((1,), (0,)), ((), ())),
        preferred_element_type=jnp.float32,
    )
    z = z + b_ref[...]
    o_ref[:, t:t + r, :] = z.reshape(bblk, r, dout)


@functools.lru_cache(maxsize=None)
def _make_project(nb, nl, dim, dout, bblk):
  return pl.pallas_call(
      functools.partial(_mm_body, nl),
      grid=(nb // bblk,),
      in_specs=[
          pl.BlockSpec((bblk, nl, dim), lambda i: (i, 0, 0)),
          pl.BlockSpec((dim, dout), lambda i: (0, 0)),
          pl.BlockSpec((1, dout), lambda i: (0, 0)),
      ],
      out_specs=pl.BlockSpec((bblk, nl, dout), lambda i: (i, 0, 0)),
      out_shape=jax.ShapeDtypeStruct((nb, nl, dout), jnp.float32),
      compiler_params=pltpu.CompilerParams(vmem_limit_bytes=100 * 1024 * 1024),
  )


def kernel(inputs, table, W, b):
  B, L = inputs.shape
  vocab, dim = table.shape
  dout = W.shape[1]
  n = B * L
  idx3d = inputs.reshape(n // (_G * _K), _K, _G).astype(jnp.int32)
  gathered = _make_gather(n // _G, vocab, dim)(idx3d, table)
  x = gathered.reshape(B, L, dim)
  return _make_project(B, L, dim, dout, 256)(x, W, b.reshape(1, dout))


# speedup vs baseline: 1.9679x; 1.9679x over previous
import functools

import jax
import jax.numpy as jnp
from jax import lax
from jax.experimental import pallas as pl
from jax.experimental.pallas import tpu as pltpu
from jax.experimental.pallas import tpu_sc as plsc

_G = 128
_K = 20


@functools.lru_cache(maxsize=None)
def _make_gather(num_groups, vocab, dim):
  info = plsc.get_sparse_core_info()
  nc, ns = info.num_cores, info.num_subcores
  nw = nc * ns
  groups_per_w = num_groups // nw
  outer = groups_per_w // _K
  assert groups_per_w % _K == 0

  mesh = plsc.VectorSubcoreMesh(core_axis_name="c", subcore_axis_name="s")

  @functools.partial(
      pl.kernel,
      mesh=mesh,
      out_type=jax.ShapeDtypeStruct((num_groups, _G, dim), jnp.float32),
      scratch_types=[
          pltpu.VMEM((_K, _G), jnp.int32),
          pltpu.VMEM((_K, _G, dim), jnp.float32),
          pltpu.SemaphoreType.DMA,
      ],
      compiler_params=pltpu.CompilerParams(use_tc_tiling_on_sc=False),
  )
  def gather_kernel(idx_hbm, table_hbm, out_hbm, idx_v, rows_v, sem):
    wid = lax.axis_index("s") * nc + lax.axis_index("c")
    tbase = wid * outer

    def step(o, carry):
      t = tbase + o
      pltpu.sync_copy(idx_hbm.at[t], idx_v)
      descs = []
      for j in range(_K):
        descs.append(
            pltpu.async_copy(table_hbm.at[idx_v.at[j]], rows_v.at[j], sem))
      for d in descs:
        d.wait()
      pltpu.sync_copy(rows_v, out_hbm.at[pl.ds(t * _K, _K)])
      return carry

    lax.fori_loop(0, outer, step, 0)

  return gather_kernel


def _mm_body(x_ref, w_ref, b_ref, o_ref):
  z = lax.dot_general(
      w_ref[...], x_ref[0],
      dimension_numbers=(((0,), (1,)), ((), ())),
      preferred_element_type=jnp.float32,
  )
  o_ref[...] = (z + b_ref[...])[None]


@functools.lru_cache(maxsize=None)
def _make_project(nl, nb, dim, dout, bblk):
  return pl.pallas_call(
      _mm_body,
      grid=(nl, nb // bblk),
      in_specs=[
          pl.BlockSpec((1, bblk, dim), lambda l, j: (l, j, 0)),
          pl.BlockSpec((dim, dout), lambda l, j: (0, 0)),
          pl.BlockSpec((dout, 1), lambda l, j: (0, 0)),
      ],
      out_specs=pl.BlockSpec((1, dout, bblk), lambda l, j: (l, 0, j)),
      out_shape=jax.ShapeDtypeStruct((nl, dout, nb), jnp.float32),
      compiler_params=pltpu.CompilerParams(vmem_limit_bytes=100 * 1024 * 1024),
  )


def kernel(inputs, table, W, b):
  B, L = inputs.shape
  vocab, dim = table.shape
  dout = W.shape[1]
  n = B * L
  idx3d = inputs.T.reshape(n // (_G * _K), _K, _G).astype(jnp.int32)
  gathered = _make_gather(n // _G, vocab, dim)(idx3d, table)
  x = gathered.reshape(L, B, dim)
  out_phys = _make_project(L, B, dim, dout, 16384)(x, W, b.reshape(dout, 1))
  return out_phys.transpose(2, 0, 1)
